# baseline (speedup 1.0000x reference)
import functools

import jax
import jax.numpy as jnp
from jax import lax
from jax.experimental import pallas as pl
from jax.experimental.pallas import tpu as pltpu
from jax.experimental.pallas import tpu_sc as plsc

E = 8
TOP_K = 2
K_POOL = 2
ROUTER_SCALE = 16.0
BIAS_SCALE = 0.5
EPS_RMS = 1e-6

TM = 256


def _norm_shared_body(x_ref, w12g_ref, w12v_ref, w3_ref, lnw_ref, nx_ref, sh_ref):
    hb = pl.program_id(1)
    x = x_ref[...]
    ms = jnp.mean(jnp.square(x), axis=1, keepdims=True)
    nx = x * jax.lax.rsqrt(ms + EPS_RMS) * lnw_ref[...]

    @pl.when(hb == 0)
    def _():
        nx_ref[...] = nx

    nxb = nx.astype(jnp.bfloat16)
    g = jax.lax.dot_general(nxb, w12g_ref[...], (((1,), (1,)), ((), ())),
                            preferred_element_type=jnp.float32)
    v = jax.lax.dot_general(nxb, w12v_ref[...], (((1,), (1,)), ((), ())),
                            preferred_element_type=jnp.float32)
    act = (g * jax.nn.sigmoid(g) * v).astype(jnp.bfloat16)
    contrib = jax.lax.dot_general(act, w3_ref[...], (((1,), (1,)), ((), ())),
                                  preferred_element_type=jnp.float32)

    @pl.when(hb == 0)
    def _():
        sh_ref[...] = contrib

    @pl.when(hb != 0)
    def _():
        sh_ref[...] += contrib


def _group_up_body(te_ref, tv_ref, xd_ref, w12g_ref, w12v_ref, act_ref):
    r = pl.program_id(1)

    @pl.when(tv_ref[r] == 1)
    def _():
        xd = xd_ref[...].astype(jnp.bfloat16)
        g = jax.lax.dot_general(xd, w12g_ref[0], (((1,), (1,)), ((), ())),
                                preferred_element_type=jnp.float32)
        v = jax.lax.dot_general(xd, w12v_ref[0], (((1,), (1,)), ((), ())),
                                preferred_element_type=jnp.float32)
        act_ref[...] = (g * jax.nn.sigmoid(g) * v).astype(jnp.bfloat16)


def _group_down_body(te_ref, tv_ref, act_ref, w3_ref, rw_ref, y_ref):
    r = pl.program_id(0)

    @pl.when(tv_ref[r] == 1)
    def _():
        act = (act_ref[...].astype(jnp.float32) * rw_ref[:, :1]).astype(jnp.bfloat16)
        y_ref[...] = jax.lax.dot_general(act, w3_ref[0], (((1,), (1,)), ((), ())),
                                         preferred_element_type=jnp.float32)


def _final_body(sh_ref, y1_ref, y2_ref, lnw_ref, out_ref):
    r = sh_ref[...] + y1_ref[...] + y2_ref[...]
    ms = jnp.mean(jnp.square(r), axis=1, keepdims=True)
    out_ref[...] = r * jax.lax.rsqrt(ms + EPS_RMS) * lnw_ref[...]


def _sc_gather(table, idx, n_rows, d, CH):
    info = plsc.get_sparse_core_info()
    nc, ns = info.num_cores, info.num_subcores
    nw = nc * ns
    per_w = n_rows // nw
    n_ch = per_w // CH
    assert per_w % CH == 0 and n_ch % 2 == 0 and per_w % 8 == 0
    mesh = plsc.VectorSubcoreMesh(core_axis_name="c", subcore_axis_name="s")

    @functools.partial(
        pl.kernel, mesh=mesh,
        out_type=jax.ShapeDtypeStruct((n_rows, d), jnp.float32),
        scratch_types=[
            pltpu.VMEM((per_w,), jnp.int32),
            pltpu.VMEM((CH, d), jnp.float32),
            pltpu.VMEM((CH, d), jnp.float32),
            pltpu.SemaphoreType.DMA,
            pltpu.SemaphoreType.DMA,
        ],
    )
    def gk(table_hbm, idx_hbm, out_hbm, idx_v, buf0, buf1, sem0, sem1):
        wid = lax.axis_index("s") * nc + lax.axis_index("c")
        base = wid * per_w
        pltpu.sync_copy(idx_hbm.at[pl.ds(base, per_w)], idx_v)
        bufs = (buf0, buf1)
        sems = (sem0, sem1)

        def issue(c, b):
            pltpu.async_copy(table_hbm.at[idx_v.at[pl.ds(c * CH, CH)]],
                             bufs[b], sems[b])

        issue(0, 0)

        def outer(c0):
            for b in range(2):
                c = c0 + b

                @pl.when(c + 1 < n_ch)
                def _():
                    issue(c + 1, 1 - b)

                pltpu.make_async_copy(
                    table_hbm.at[pl.ds(0, CH)], bufs[b], sems[b]).wait()
                pltpu.sync_copy(bufs[b], out_hbm.at[pl.ds(base + c * CH, CH)])

        lax.fori_loop(0, n_ch // 2, lambda i, _: (outer(2 * i), 0)[1], 0)

    return gk(table, idx)


def kernel(x, router_weights, shared_w12, shared_w3, expert_w12, expert_w3,
           ln_pre_w, ln_post_w, expert_load):
    B, S, Denc = x.shape
    D = Denc * K_POOL
    xf = x.reshape(B, S // K_POOL, D).reshape(-1, D)
    N = xf.shape[0]
    HS = shared_w12.shape[0] // 2
    HR = expert_w12.shape[1] // 2
    OUT = shared_w3.shape[0]
    R = TOP_K * N + E * TM
    n_tiles = R // TM

    var = jnp.mean(jnp.square(xf), axis=-1, keepdims=True)
    norm_x_r = xf * jax.lax.rsqrt(var + EPS_RMS) * ln_pre_w
    n1 = jnp.linalg.norm(norm_x_r, axis=-1, keepdims=True)
    input_normed = norm_x_r / jnp.maximum(n1, 1e-12)
    n2 = jnp.linalg.norm(router_weights, axis=-1, keepdims=True)
    router_normed = router_weights / jnp.maximum(n2, 1e-12)
    logits = input_normed @ router_normed.T * ROUTER_SCALE
    probs = jax.nn.sigmoid(logits)
    choice = probs - BIAS_SCALE * expert_load
    _, idx = jax.lax.top_k(choice, TOP_K)
    tkw = jnp.take_along_axis(probs, idx, axis=-1)
    tkw = tkw / (jnp.sum(tkw, axis=-1, keepdims=True) + 1e-20)

    flat_ids = idx.reshape(-1).astype(jnp.int32)
    perm = jnp.argsort(flat_ids, stable=True).astype(jnp.int32)
    sorted_ids = flat_ids[perm]
    token_of_sorted = (perm // TOP_K).astype(jnp.int32)
    counts = jnp.zeros((E,), jnp.int32).at[flat_ids].add(1)
    start = jnp.concatenate([jnp.zeros((1,), jnp.int32),
                             jnp.cumsum(counts)[:-1].astype(jnp.int32)])
    pc = ((counts + TM - 1) // TM) * TM
    cum_pc = jnp.cumsum(pc).astype(jnp.int32)
    offsets = jnp.concatenate([jnp.zeros((1,), jnp.int32), cum_pc[:-1]])
    j = jnp.arange(TOP_K * N, dtype=jnp.int32)
    pos_sorted = offsets[sorted_ids] + (j - start[sorted_ids])
    row_token = jnp.zeros((R,), jnp.int32).at[pos_sorted].set(token_of_sorted)
    inv_pos = jnp.zeros((TOP_K * N,), jnp.int32).at[perm].set(pos_sorted)
    pos_cat = inv_pos.reshape(N, TOP_K).T.reshape(-1)
    row_w = jnp.zeros((R,), jnp.float32).at[pos_sorted].set(tkw.reshape(-1)[perm])
    row_w2 = jnp.tile(row_w[:, None], (1, 128))
    tile_start = jnp.arange(n_tiles, dtype=jnp.int32) * TM
    tile_expert = jnp.minimum(
        jnp.searchsorted(cum_pc, tile_start, side="right").astype(jnp.int32), E - 1)
    tile_valid = (tile_start < cum_pc[E - 1]).astype(jnp.int32)

    sw12_b = shared_w12.astype(jnp.bfloat16)
    sw3_b = shared_w3.astype(jnp.bfloat16)
    ew12_b = expert_w12.astype(jnp.bfloat16)
    ew3_b = expert_w3.astype(jnp.bfloat16)

    TA = min(512, N)
    HBA = min(512, HS)
    n_hba = HS // HBA
    nx, sh = pl.pallas_call(
        _norm_shared_body,
        grid=(N // TA, n_hba),
        in_specs=[
            pl.BlockSpec((TA, D), lambda t, hb: (t, 0)),
            pl.BlockSpec((HBA, D), lambda t, hb: (hb, 0)),
            pl.BlockSpec((HBA, D), lambda t, hb, o=n_hba: (hb + o, 0)),
            pl.BlockSpec((OUT, HBA), lambda t, hb: (0, hb)),
            pl.BlockSpec((1, D), lambda t, hb: (0, 0)),
        ],
        out_specs=[
            pl.BlockSpec((TA, D), lambda t, hb: (t, 0)),
            pl.BlockSpec((TA, OUT), lambda t, hb: (t, 0)),
        ],
        out_shape=[
            jax.ShapeDtypeStruct((N, D), jnp.float32),
            jax.ShapeDtypeStruct((N, OUT), jnp.float32),
        ],
        compiler_params=pltpu.CompilerParams(
            dimension_semantics=("parallel", "arbitrary")),
    )(xf, sw12_b, sw12_b, sw3_b, ln_pre_w.reshape(1, D))

    xd = _sc_gather(nx, row_token, R, D, 16)

    nh2 = 2
    HB1 = HR // nh2
    act = pl.pallas_call(
        _group_up_body,
        grid_spec=pltpu.PrefetchScalarGridSpec(
            num_scalar_prefetch=2,
            grid=(nh2, n_tiles),
            in_specs=[
                pl.BlockSpec((TM, D), lambda hb, r, te, tv: (r, 0)),
                pl.BlockSpec((1, HB1, D), lambda hb, r, te, tv: (te[r], hb, 0)),
                pl.BlockSpec((1, HB1, D),
                             lambda hb, r, te, tv, o=nh2: (te[r], hb + o, 0)),
            ],
            out_specs=pl.BlockSpec((TM, HB1), lambda hb, r, te, tv: (r, hb)),
        ),
        out_shape=jax.ShapeDtypeStruct((R, HR), jnp.bfloat16),
        compiler_params=pltpu.CompilerParams(
            dimension_semantics=("arbitrary", "arbitrary")),
    )(tile_expert, tile_valid, xd, ew12_b, ew12_b)

    y = pl.pallas_call(
        _group_down_body,
        grid_spec=pltpu.PrefetchScalarGridSpec(
            num_scalar_prefetch=2,
            grid=(n_tiles,),
            in_specs=[
                pl.BlockSpec((TM, HR), lambda r, te, tv: (r, 0)),
                pl.BlockSpec((1, OUT, HR), lambda r, te, tv: (te[r], 0, 0)),
                pl.BlockSpec((TM, 128), lambda r, te, tv: (r, 0)),
            ],
            out_specs=pl.BlockSpec((TM, OUT), lambda r, te, tv: (r, 0)),
        ),
        out_shape=jax.ShapeDtypeStruct((R, OUT), jnp.float32),
        compiler_params=pltpu.CompilerParams(
            dimension_semantics=("arbitrary",)),
    )(tile_expert, tile_valid, act, ew3_b, row_w2)

    y12 = _sc_gather(y, pos_cat, TOP_K * N, OUT, 16)

    TF = min(512, N)
    nf = N // TF
    out = pl.pallas_call(
        _final_body,
        grid=(nf,),
        in_specs=[
            pl.BlockSpec((TF, OUT), lambda t: (t, 0)),
            pl.BlockSpec((TF, OUT), lambda t: (t, 0)),
            pl.BlockSpec((TF, OUT), lambda t, o=nf: (t + o, 0)),
            pl.BlockSpec((1, OUT), lambda t: (0, 0)),
        ],
        out_specs=pl.BlockSpec((TF, OUT), lambda t: (t, 0)),
        out_shape=jax.ShapeDtypeStruct((N, OUT), jnp.float32),
        compiler_params=pltpu.CompilerParams(
            dimension_semantics=("parallel",)),
    )(sh, y12, y12, ln_post_w.reshape(1, OUT))

    aux = jnp.asarray(0.0, dtype=x.dtype)
    return out.reshape(B, S // K_POOL, OUT), aux


# speedup vs baseline: 1.2649x; 1.2649x over previous
import functools

import jax
import jax.numpy as jnp
from jax import lax
from jax.experimental import pallas as pl
from jax.experimental.pallas import tpu as pltpu
from jax.experimental.pallas import tpu_sc as plsc

E = 8
TOP_K = 2
K_POOL = 2
ROUTER_SCALE = 16.0
BIAS_SCALE = 0.5
EPS_RMS = 1e-6

TM = 256


def _norm_shared_body(x_ref, w12g_ref, w12v_ref, w3_ref, lnw_ref, sh_ref):
    hb = pl.program_id(1)
    x = x_ref[...]
    ms = jnp.mean(jnp.square(x), axis=1, keepdims=True)
    nx = x * jax.lax.rsqrt(ms + EPS_RMS) * lnw_ref[...]
    nxb = nx.astype(jnp.bfloat16)
    g = jax.lax.dot_general(nxb, w12g_ref[...].astype(jnp.bfloat16),
                            (((1,), (1,)), ((), ())),
                            preferred_element_type=jnp.float32)
    v = jax.lax.dot_general(nxb, w12v_ref[...].astype(jnp.bfloat16),
                            (((1,), (1,)), ((), ())),
                            preferred_element_type=jnp.float32)
    act = (g * jax.nn.sigmoid(g) * v).astype(jnp.bfloat16)
    contrib = jax.lax.dot_general(act, w3_ref[...].astype(jnp.bfloat16),
                                  (((1,), (1,)), ((), ())),
                                  preferred_element_type=jnp.float32)

    @pl.when(hb == 0)
    def _():
        sh_ref[...] = contrib

    @pl.when(hb != 0)
    def _():
        sh_ref[...] += contrib


def _group_up_body(te_ref, tv_ref, xd_ref, w12g_ref, w12v_ref, lnw_ref, act_ref):
    r = pl.program_id(1)

    @pl.when(tv_ref[r] == 1)
    def _():
        xd = xd_ref[...]
        ms = jnp.mean(jnp.square(xd), axis=1, keepdims=True)
        nx = xd * jax.lax.rsqrt(ms + EPS_RMS) * lnw_ref[...]
        nxb = nx.astype(jnp.bfloat16)
        g = jax.lax.dot_general(nxb, w12g_ref[0].astype(jnp.bfloat16),
                                (((1,), (1,)), ((), ())),
                                preferred_element_type=jnp.float32)
        v = jax.lax.dot_general(nxb, w12v_ref[0].astype(jnp.bfloat16),
                                (((1,), (1,)), ((), ())),
                                preferred_element_type=jnp.float32)
        act_ref[...] = (g * jax.nn.sigmoid(g) * v).astype(jnp.bfloat16)


def _group_down_body(te_ref, tv_ref, act_ref, w3_ref, rw_ref, y_ref):
    r = pl.program_id(0)

    @pl.when(tv_ref[r] == 1)
    def _():
        act = (act_ref[...].astype(jnp.float32) * rw_ref[:, :1]).astype(jnp.bfloat16)
        y_ref[...] = jax.lax.dot_general(act, w3_ref[0].astype(jnp.bfloat16),
                                         (((1,), (1,)), ((), ())),
                                         preferred_element_type=jnp.float32)


def _final_body(sh_ref, y1_ref, y2_ref, lnw_ref, out_ref):
    r = sh_ref[...] + y1_ref[...] + y2_ref[...]
    ms = jnp.mean(jnp.square(r), axis=1, keepdims=True)
    out_ref[...] = r * jax.lax.rsqrt(ms + EPS_RMS) * lnw_ref[...]


def _sc_gather(table, idx, n_rows, d, CH):
    info = plsc.get_sparse_core_info()
    nc, ns = info.num_cores, info.num_subcores
    nw = nc * ns
    per_w = n_rows // nw
    n_ch = per_w // CH
    assert per_w % CH == 0 and n_ch % 2 == 0 and per_w % 8 == 0
    mesh = plsc.VectorSubcoreMesh(core_axis_name="c", subcore_axis_name="s")

    @functools.partial(
        pl.kernel, mesh=mesh,
        out_type=jax.ShapeDtypeStruct((n_rows, d), jnp.float32),
        scratch_types=[
            pltpu.VMEM((per_w,), jnp.int32),
            pltpu.VMEM((CH, d), jnp.float32),
            pltpu.VMEM((CH, d), jnp.float32),
            pltpu.SemaphoreType.DMA,
            pltpu.SemaphoreType.DMA,
        ],
    )
    def gk(table_hbm, idx_hbm, out_hbm, idx_v, buf0, buf1, sem0, sem1):
        wid = lax.axis_index("s") * nc + lax.axis_index("c")
        base = wid * per_w
        pltpu.sync_copy(idx_hbm.at[pl.ds(base, per_w)], idx_v)
        bufs = (buf0, buf1)
        sems = (sem0, sem1)

        def issue(c, b):
            pltpu.async_copy(table_hbm.at[idx_v.at[pl.ds(c * CH, CH)]],
                             bufs[b], sems[b])

        issue(0, 0)

        def outer(c0):
            for b in range(2):
                c = c0 + b

                @pl.when(c + 1 < n_ch)
                def _():
                    issue(c + 1, 1 - b)

                pltpu.make_async_copy(
                    table_hbm.at[pl.ds(0, CH)], bufs[b], sems[b]).wait()
                pltpu.sync_copy(bufs[b], out_hbm.at[pl.ds(base + c * CH, CH)])

        lax.fori_loop(0, n_ch // 2, lambda i, _: (outer(2 * i), 0)[1], 0)

    return gk(table, idx)


def kernel(x, router_weights, shared_w12, shared_w3, expert_w12, expert_w3,
           ln_pre_w, ln_post_w, expert_load):
    B, S, Denc = x.shape
    D = Denc * K_POOL
    xf = x.reshape(B, S // K_POOL, D).reshape(-1, D)
    N = xf.shape[0]
    HS = shared_w12.shape[0] // 2
    HR = expert_w12.shape[1] // 2
    OUT = shared_w3.shape[0]
    R = TOP_K * N + E * TM
    n_tiles = R // TM

    var = jnp.mean(jnp.square(xf), axis=-1, keepdims=True)
    norm_x_r = xf * jax.lax.rsqrt(var + EPS_RMS) * ln_pre_w
    n1 = jnp.linalg.norm(norm_x_r, axis=-1, keepdims=True)
    input_normed = norm_x_r / jnp.maximum(n1, 1e-12)
    n2 = jnp.linalg.norm(router_weights, axis=-1, keepdims=True)
    router_normed = router_weights / jnp.maximum(n2, 1e-12)
    logits = input_normed @ router_normed.T * ROUTER_SCALE
    probs = jax.nn.sigmoid(logits)
    choice = probs - BIAS_SCALE * expert_load
    _, idx = jax.lax.top_k(choice, TOP_K)
    tkw = jnp.take_along_axis(probs, idx, axis=-1)
    tkw = tkw / (jnp.sum(tkw, axis=-1, keepdims=True) + 1e-20)

    flat_ids = idx.reshape(-1).astype(jnp.int32)
    perm = jnp.argsort(flat_ids, stable=True).astype(jnp.int32)
    sorted_ids = flat_ids[perm]
    token_of_sorted = (perm // TOP_K).astype(jnp.int32)
    counts = jnp.zeros((E,), jnp.int32).at[flat_ids].add(1)
    start = jnp.concatenate([jnp.zeros((1,), jnp.int32),
                             jnp.cumsum(counts)[:-1].astype(jnp.int32)])
    pc = ((counts + TM - 1) // TM) * TM
    cum_pc = jnp.cumsum(pc).astype(jnp.int32)
    offsets = jnp.concatenate([jnp.zeros((1,), jnp.int32), cum_pc[:-1]])
    j = jnp.arange(TOP_K * N, dtype=jnp.int32)
    pos_sorted = offsets[sorted_ids] + (j - start[sorted_ids])
    row_token = jnp.zeros((R,), jnp.int32).at[pos_sorted].set(token_of_sorted)
    inv_pos = jnp.zeros((TOP_K * N,), jnp.int32).at[perm].set(pos_sorted)
    pos_cat = inv_pos.reshape(N, TOP_K).T.reshape(-1)
    row_w = jnp.zeros((R,), jnp.float32).at[pos_sorted].set(tkw.reshape(-1)[perm])
    row_w2 = jnp.tile(row_w[:, None], (1, 128))
    tile_start = jnp.arange(n_tiles, dtype=jnp.int32) * TM
    tile_expert = jnp.minimum(
        jnp.searchsorted(cum_pc, tile_start, side="right").astype(jnp.int32), E - 1)
    tile_valid = (tile_start < cum_pc[E - 1]).astype(jnp.int32)

    TA = min(512, N)
    HBA = min(512, HS)
    n_hba = HS // HBA
    sh = pl.pallas_call(
        _norm_shared_body,
        grid=(N // TA, n_hba),
        in_specs=[
            pl.BlockSpec((TA, D), lambda t, hb: (t, 0)),
            pl.BlockSpec((HBA, D), lambda t, hb: (hb, 0)),
            pl.BlockSpec((HBA, D), lambda t, hb, o=n_hba: (hb + o, 0)),
            pl.BlockSpec((OUT, HBA), lambda t, hb: (0, hb)),
            pl.BlockSpec((1, D), lambda t, hb: (0, 0)),
        ],
        out_specs=pl.BlockSpec((TA, OUT), lambda t, hb: (t, 0)),
        out_shape=jax.ShapeDtypeStruct((N, OUT), jnp.float32),
        compiler_params=pltpu.CompilerParams(
            dimension_semantics=("parallel", "arbitrary")),
    )(xf, shared_w12, shared_w12, shared_w3, ln_pre_w.reshape(1, D))

    xd = _sc_gather(xf, row_token, R, D, 16)

    nh2 = 2
    HB1 = HR // nh2
    act = pl.pallas_call(
        _group_up_body,
        grid_spec=pltpu.PrefetchScalarGridSpec(
            num_scalar_prefetch=2,
            grid=(nh2, n_tiles),
            in_specs=[
                pl.BlockSpec((TM, D), lambda hb, r, te, tv: (r, 0)),
                pl.BlockSpec((1, HB1, D), lambda hb, r, te, tv: (te[r], hb, 0)),
                pl.BlockSpec((1, HB1, D),
                             lambda hb, r, te, tv, o=nh2: (te[r], hb + o, 0)),
                pl.BlockSpec((1, D), lambda hb, r, te, tv: (0, 0)),
            ],
            out_specs=pl.BlockSpec((TM, HB1), lambda hb, r, te, tv: (r, hb)),
        ),
        out_shape=jax.ShapeDtypeStruct((R, HR), jnp.bfloat16),
        compiler_params=pltpu.CompilerParams(
            dimension_semantics=("arbitrary", "arbitrary")),
    )(tile_expert, tile_valid, xd, expert_w12, expert_w12, ln_pre_w.reshape(1, D))

    y = pl.pallas_call(
        _group_down_body,
        grid_spec=pltpu.PrefetchScalarGridSpec(
            num_scalar_prefetch=2,
            grid=(n_tiles,),
            in_specs=[
                pl.BlockSpec((TM, HR), lambda r, te, tv: (r, 0)),
                pl.BlockSpec((1, OUT, HR), lambda r, te, tv: (te[r], 0, 0)),
                pl.BlockSpec((TM, 128), lambda r, te, tv: (r, 0)),
            ],
            out_specs=pl.BlockSpec((TM, OUT), lambda r, te, tv: (r, 0)),
        ),
        out_shape=jax.ShapeDtypeStruct((R, OUT), jnp.float32),
        compiler_params=pltpu.CompilerParams(
            dimension_semantics=("arbitrary",)),
    )(tile_expert, tile_valid, act, expert_w3, row_w2)

    y12 = _sc_gather(y, pos_cat, TOP_K * N, OUT, 16)

    TF = min(512, N)
    nf = N // TF
    out = pl.pallas_call(
        _final_body,
        grid=(nf,),
        in_specs=[
            pl.BlockSpec((TF, OUT), lambda t: (t, 0)),
            pl.BlockSpec((TF, OUT), lambda t: (t, 0)),
            pl.BlockSpec((TF, OUT), lambda t, o=nf: (t + o, 0)),
            pl.BlockSpec((1, OUT), lambda t: (0, 0)),
        ],
        out_specs=pl.BlockSpec((TF, OUT), lambda t: (t, 0)),
        out_shape=jax.ShapeDtypeStruct((N, OUT), jnp.float32),
        compiler_params=pltpu.CompilerParams(
            dimension_semantics=("parallel",)),
    )(sh, y12, y12, ln_post_w.reshape(1, OUT))

    aux = jnp.asarray(0.0, dtype=x.dtype)
    return out.reshape(B, S // K_POOL, OUT), aux
